# baseline (speedup 1.0000x reference)
import functools

import jax
import jax.numpy as jnp
from jax import lax
from jax.experimental import pallas as pl
from jax.experimental.pallas import tpu as pltpu
from jax.experimental.pallas import tpu_sc as plsc

E = 8
T = 2048
H = 2048
TB = 128
KS = 512
NW = 32
CH = 2
NTOK = KS // NW
NCHUNK = NTOK // CH
NV = H // 16


def _rsqrt_newton(v):
    i = lax.bitcast_convert_type(v, jnp.int32)
    i = jnp.int32(0x5F3759DF) - (i >> 1)
    y = lax.bitcast_convert_type(i, jnp.float32)
    for _ in range(3):
        y = y * (1.5 - 0.5 * v * y * y)
    return y


def _sc_body(active, scale_b, token, resid, nw, epsv, hs_out, or_out,
             act_buf, tok_buf, res_buf, or_buf, hs_buf, scl_buf, nw_buf,
             eps_buf, ssq_buf, sem_in0, sem_in1, sem_out0, sem_out1, sem_pro):
    wid = lax.axis_index("c") * 16 + lax.axis_index("s")
    base = wid * NTOK

    pltpu.async_copy(nw, nw_buf, sem_pro).wait()
    pltpu.async_copy(epsv, eps_buf, sem_pro).wait()
    pltpu.async_copy(scale_b.at[pl.ds(base, NTOK)], scl_buf, sem_pro).wait()

    in_sems = (sem_in0, sem_in1)
    out_sems = (sem_out0, sem_out1)

    def issue_in(c, p):
        t0 = base + c * CH
        hs = []
        for e in range(E):
            hs.append(pltpu.async_copy(active.at[e, pl.ds(t0, CH)],
                                       act_buf.at[p, e], in_sems[p]))
        hs.append(pltpu.async_copy(token.at[pl.ds(t0, CH)],
                                   tok_buf.at[p], in_sems[p]))
        hs.append(pltpu.async_copy(resid.at[pl.ds(t0, CH)],
                                   res_buf.at[p], in_sems[p]))
        return hs

    pending_out = [None, None]
    handles = issue_in(0, 0)
    for c in range(NCHUNK):
        p = c % 2
        nxt = issue_in(c + 1, 1 - p) if c + 1 < NCHUNK else None
        for h in handles:
            h.wait()
        if pending_out[p] is not None:
            for h in pending_out[p]:
                h.wait()
        eps_vec = eps_buf[...]
        for t in range(CH):
            tl = c * CH + t
            svecs = [scl_buf[tl, e] for e in range(E)]

            def body(v, ssq, _t=t, _p=p, _svecs=svecs):
                sl = pl.ds(v * 16, 16)
                a = tok_buf[_p, _t, sl] + res_buf[_p, _t, sl]
                for e in range(E):
                    a = a + act_buf[_p, e, _t, sl] * _svecs[e]
                or_buf[_p, _t, sl] = a
                return ssq + a * a

            ssq = lax.fori_loop(0, NV, body, jnp.zeros((16,), jnp.float32),
                                unroll=8)
            tot = ssq[0]
            for i in range(1, 16):
                tot = tot + ssq[i]
            var = tot * (1.0 / H)
            rvec = _rsqrt_newton(
                lax.broadcast_in_dim(var, (16,), ()) + eps_vec)

            def body2(v, carry, _t=t, _p=p, _rvec=rvec):
                sl = pl.ds(v * 16, 16)
                hs_buf[_p, _t, sl] = or_buf[_p, _t, sl] * _rvec * nw_buf[sl]
                return carry

            lax.fori_loop(0, NV, body2, 0, unroll=8)
        t0 = base + c * CH
        pending_out[p] = [
            pltpu.async_copy(or_buf.at[p], or_out.at[pl.ds(t0, CH)],
                             out_sems[p]),
            pltpu.async_copy(hs_buf.at[p], hs_out.at[pl.ds(t0, CH)],
                             out_sems[p]),
        ]
        handles = nxt
    for p in range(2):
        if pending_out[p] is not None:
            for h in pending_out[p]:
                h.wait()


def _sc_run(active, scale_b, token, resid, nw, epsv):
    mesh = plsc.VectorSubcoreMesh(core_axis_name="c", subcore_axis_name="s")
    f = functools.partial(
        pl.kernel,
        mesh=mesh,
        out_type=[
            jax.ShapeDtypeStruct((KS, H), jnp.float32),
            jax.ShapeDtypeStruct((KS, H), jnp.float32),
        ],
        scratch_types=[
            pltpu.VMEM((2, E, CH, H), jnp.float32),
            pltpu.VMEM((2, CH, H), jnp.float32),
            pltpu.VMEM((2, CH, H), jnp.float32),
            pltpu.VMEM((2, CH, H), jnp.float32),
            pltpu.VMEM((2, CH, H), jnp.float32),
            pltpu.VMEM((NTOK, E, 16), jnp.float32),
            pltpu.VMEM((H,), jnp.float32),
            pltpu.VMEM((16,), jnp.float32),
            pltpu.VMEM((16,), jnp.float32),
            pltpu.SemaphoreType.DMA,
            pltpu.SemaphoreType.DMA,
            pltpu.SemaphoreType.DMA,
            pltpu.SemaphoreType.DMA,
            pltpu.SemaphoreType.DMA,
        ],
    )(_sc_body)
    return f(active, scale_b, token, resid, nw, epsv)


def _tc_body(eps_ref, scale_ref, active_ref, token_ref, resid_ref, nw_ref,
             hs_ref, outres_ref):
    acc = token_ref[...] + resid_ref[...]
    for e in range(E):
        acc = acc + active_ref[e] * scale_ref[0, :, e][:, None]
    outres_ref[...] = acc
    var = jnp.mean(acc * acc, axis=-1, keepdims=True)
    hs_ref[...] = acc * jax.lax.rsqrt(var + eps_ref[0]) * nw_ref[...]


def kernel(residual, norm_weight, device_num_experts, scale_input,
           active_experts_token_input, token_input, eps):
    del device_num_experts
    eps_arr = jnp.asarray(eps, dtype=jnp.float32).reshape(1)
    nw = norm_weight.reshape(1, H)
    scale_t = scale_input.T.reshape(T // TB, TB, E)

    eps_vec = jnp.broadcast_to(jnp.asarray(eps, jnp.float32), (16,))
    scale_b = jnp.broadcast_to(scale_input.T[:KS, :, None], (KS, E, 16))
    sc_hs, sc_or = _sc_run(active_experts_token_input, scale_b, token_input,
                           residual, norm_weight, eps_vec)

    koff = KS // TB
    nblk = (T - KS) // TB
    tc_hs, tc_or = pl.pallas_call(
        _tc_body,
        grid=(nblk,),
        in_specs=[
            pl.BlockSpec((1,), lambda i: (0,)),
            pl.BlockSpec((1, TB, E), lambda i: (i + koff, 0, 0)),
            pl.BlockSpec((E, TB, H), lambda i: (0, i + koff, 0)),
            pl.BlockSpec((TB, H), lambda i: (i + koff, 0)),
            pl.BlockSpec((TB, H), lambda i: (i + koff, 0)),
            pl.BlockSpec((1, H), lambda i: (0, 0)),
        ],
        out_specs=[
            pl.BlockSpec((TB, H), lambda i: (i + koff, 0)),
            pl.BlockSpec((TB, H), lambda i: (i + koff, 0)),
        ],
        out_shape=[
            jax.ShapeDtypeStruct((T, H), jnp.float32),
            jax.ShapeDtypeStruct((T, H), jnp.float32),
        ],
    )(eps_arr, scale_t, active_experts_token_input, token_input, residual, nw)

    hs = lax.dynamic_update_slice(tc_hs, sc_hs, (0, 0))
    outres = lax.dynamic_update_slice(tc_or, sc_or, (0, 0))
    return hs, outres


# speedup vs baseline: 1.5043x; 1.5043x over previous
import jax
import jax.numpy as jnp
from jax.experimental import pallas as pl

E = 8
T = 2048
H = 2048
TB = 128


def _fused_body(eps_ref, scale_ref, active_ref, token_ref, resid_ref, nw_ref,
                hs_ref, outres_ref):
    acc = token_ref[...] + resid_ref[...]
    scol = scale_ref[...].T
    for e in range(E):
        acc = acc + active_ref[e] * scol[:, e][:, None]
    outres_ref[...] = acc
    var = jnp.mean(acc * acc, axis=-1, keepdims=True)
    hs_ref[...] = acc * jax.lax.rsqrt(var + eps_ref[0]) * nw_ref[...]


def kernel(residual, norm_weight, device_num_experts, scale_input,
           active_experts_token_input, token_input, eps):
    del device_num_experts
    eps_arr = jnp.asarray(eps, dtype=jnp.float32).reshape(1)
    nw = norm_weight.reshape(1, H)

    return pl.pallas_call(
        _fused_body,
        grid=(T // TB,),
        in_specs=[
            pl.BlockSpec((1,), lambda i: (0,)),
            pl.BlockSpec((E, TB), lambda i: (0, i)),
            pl.BlockSpec((E, TB, H), lambda i: (0, i, 0)),
            pl.BlockSpec((TB, H), lambda i: (i, 0)),
            pl.BlockSpec((TB, H), lambda i: (i, 0)),
            pl.BlockSpec((1, H), lambda i: (0, 0)),
        ],
        out_specs=[
            pl.BlockSpec((TB, H), lambda i: (i, 0)),
            pl.BlockSpec((TB, H), lambda i: (i, 0)),
        ],
        out_shape=[
            jax.ShapeDtypeStruct((T, H), jnp.float32),
            jax.ShapeDtypeStruct((T, H), jnp.float32),
        ],
    )(eps_arr, scale_input, active_experts_token_input, token_input,
      residual, nw)
